# Initial kernel scaffold; baseline (speedup 1.0000x reference)
import jax
import jax.numpy as jnp
from jax.experimental import pallas as pl


def kernel(pointcloud, params):
    raise NotImplementedError("write your pallas kernel here")


# speedup vs baseline: 1.9955x; 1.9955x over previous
import functools

import jax
import jax.numpy as jnp
from jax.experimental import pallas as pl

F32 = jnp.float32


def _fiota(shape, dim):
    return jax.lax.broadcasted_iota(jnp.int32, shape, dim).astype(F32)


def _fps_kernel(xt_ref, out_ref, *, npoint, n):
    b = xt_ref.shape[0]
    niota = _fiota((b, n), 1)
    piota = jax.lax.broadcasted_iota(jnp.int32, (b, npoint), 1)
    x0 = xt_ref[:, 0, :]
    x1 = xt_ref[:, 1, :]
    x2 = xt_ref[:, 2, :]

    def body(i, carry):
        dist, oh, cx, cy, cz = carry
        c0 = jnp.sum(x0 * oh, axis=1, keepdims=True)
        c1 = jnp.sum(x1 * oh, axis=1, keepdims=True)
        c2 = jnp.sum(x2 * oh, axis=1, keepdims=True)
        sel = (piota == i).astype(F32)
        cx = cx + c0 * sel
        cy = cy + c1 * sel
        cz = cz + c2 * sel
        d = (x0 - c0) ** 2 + (x1 - c1) ** 2 + (x2 - c2) ** 2
        dist = jnp.minimum(dist, d)
        m = jnp.max(dist, axis=1, keepdims=True)
        idx = jnp.min(jnp.where(dist == m, niota, float(n)), axis=1,
                      keepdims=True)
        oh = (niota == idx).astype(F32)
        return dist, oh, cx, cy, cz

    dist0 = jnp.full((b, n), 1e10, F32)
    oh0 = (niota == 0.0).astype(F32)
    z = jnp.zeros((b, npoint), F32)
    _, _, cx, cy, cz = jax.lax.fori_loop(0, npoint, body,
                                         (dist0, oh0, z, z, z))
    out_ref[:, 0, :] = cx
    out_ref[:, 1, :] = cy
    out_ref[:, 2, :] = cz


def _fps(xt, npoint):
    b, _, n = xt.shape
    return pl.pallas_call(
        functools.partial(_fps_kernel, npoint=npoint, n=n),
        out_shape=jax.ShapeDtypeStruct((b, 3, npoint), F32),
        in_specs=[pl.BlockSpec((b, 3, n), lambda: (0, 0, 0))],
        out_specs=pl.BlockSpec((b, 3, npoint), lambda: (0, 0, 0)),
    )(xt)


def _mlp3(x, w1, s1, b1, w2, s2, b2, w3, s3, b3):
    for w, s, b in ((w1, s1, b1), (w2, s2, b2), (w3, s3, b3)):
        x = jnp.dot(x.astype(jnp.bfloat16), w[...],
                    preferred_element_type=F32)
        x = jnp.maximum(x * s[...] + b[...], 0.0)
    return x


def _sa_kernel(xt_ref, xf_ref, nx_ref, w1_ref, s1_ref, b1_ref, w2_ref,
               s2_ref, b2_ref, w3_ref, s3_ref, b3_ref, o_ref,
               *, r2, k, n, sb, cw):
    nc = n // 128
    xt = xt_ref[0]
    nx = nx_ref[0]
    aa = nx[:, 0:1] ** 2 + nx[:, 1:2] ** 2 + nx[:, 2:3] ** 2
    bb = xt[0:1, :] ** 2 + xt[1:2, :] ** 2 + xt[2:3, :] ** 2
    ab = jnp.dot(nx.astype(jnp.bfloat16), xt.astype(jnp.bfloat16),
                 preferred_element_type=F32)
    d2 = aa + bb - 2.0 * ab
    m = (d2 < r2).astype(F32)

    ii = _fiota((128, 128), 0)
    jj = _fiota((128, 128), 1)
    tri = (ii <= jj).astype(F32)
    incl, tots = [], []
    for c in range(nc):
        mc = m[:, 128 * c:128 * (c + 1)]
        ic = jnp.dot(mc, tri, preferred_element_type=F32)
        incl.append(ic)
        tots.append(ic[:, 127:128])
    totals = jnp.concatenate(tots, axis=1)
    si = _fiota((nc, nc), 0)
    sj = _fiota((nc, nc), 1)
    stri = (si < sj).astype(F32)
    offs = jnp.dot(totals, stri, preferred_element_type=F32)
    rank = jnp.concatenate(
        [incl[c] - m[:, 128 * c:128 * (c + 1)] + offs[:, c:c + 1]
         for c in range(nc)], axis=1)
    counts = offs[:, nc - 1:nc] + totals[:, nc - 1:nc]

    kio = _fiota((k, 1), 0)
    cp = xf_ref.shape[2]
    nxpad = jnp.concatenate([nx, jnp.zeros((sb, cp - 3), F32)], axis=1)
    g_rows = []
    for s in range(sb):
        cnt = counts[s:s + 1, 0:1]
        keff = jnp.minimum(kio, cnt - 1.0)
        rs = rank[s:s + 1, :]
        ms = m[s:s + 1, :]
        g = jnp.zeros((k, cp), F32)
        for c0 in range(0, n, cw):
            e = (rs[:, c0:c0 + cw] == keff).astype(F32) * ms[:, c0:c0 + cw]
            if c0 == 0:
                oh0 = (_fiota((1, cw), 1)
                       == 0.0).astype(F32)
                e = jnp.where(cnt > 0.0, e, oh0)
            else:
                e = jnp.where(cnt > 0.0, e, 0.0)
            g = g + jnp.dot(e, xf_ref[0, c0:c0 + cw, :],
                            preferred_element_type=F32,
                            precision=jax.lax.Precision.HIGHEST)
        g_rows.append(g - nxpad[s:s + 1, :])
    gall = jnp.concatenate(g_rows, axis=0)

    a = _mlp3(gall, w1_ref, s1_ref, b1_ref, w2_ref, s2_ref, b2_ref,
              w3_ref, s3_ref, b3_ref)
    for s in range(sb):
        o_ref[0, s:s + 1, :] = jnp.max(a[s * k:(s + 1) * k, :], axis=0,
                                       keepdims=True)


def _sa(xt, xf, nx, wts, r2, k, sb, cw):
    b, _, n = xt.shape
    s_tot = nx.shape[1]
    cp = xf.shape[2]
    (w1, s1, b1), (w2, s2, b2), (w3, s3, b3) = wts
    c3 = w3.shape[1]
    wspecs = [pl.BlockSpec(a.shape, lambda bi, st: (0, 0))
              for a in (w1, s1, b1, w2, s2, b2, w3, s3, b3)]
    return pl.pallas_call(
        functools.partial(_sa_kernel, r2=r2, k=k, n=n, sb=sb, cw=cw),
        grid=(b, s_tot // sb),
        out_shape=jax.ShapeDtypeStruct((b, s_tot, c3), F32),
        in_specs=[
            pl.BlockSpec((1, 3, n), lambda bi, st: (bi, 0, 0)),
            pl.BlockSpec((1, n, cp), lambda bi, st: (bi, 0, 0)),
            pl.BlockSpec((1, sb, 3), lambda bi, st: (bi, st, 0)),
        ] + wspecs,
        out_specs=pl.BlockSpec((1, sb, c3), lambda bi, st: (bi, st, 0)),
    )(xt, xf, nx, w1, s1, b1, w2, s2, b2, w3, s3, b3)


def _ga_kernel(xf_ref, w1_ref, s1_ref, b1_ref, w2_ref, s2_ref, b2_ref,
               w3_ref, s3_ref, b3_ref, o_ref):
    a = _mlp3(xf_ref[0], w1_ref, s1_ref, b1_ref, w2_ref, s2_ref, b2_ref,
              w3_ref, s3_ref, b3_ref)
    o_ref[0] = jnp.max(a, axis=0, keepdims=True)


def _ga(xf, wts):
    b, n, cp = xf.shape
    (w1, s1, b1), (w2, s2, b2), (w3, s3, b3) = wts
    c3 = w3.shape[1]
    wspecs = [pl.BlockSpec(a.shape, lambda bi: (0, 0))
              for a in (w1, s1, b1, w2, s2, b2, w3, s3, b3)]
    out = pl.pallas_call(
        _ga_kernel,
        grid=(b,),
        out_shape=jax.ShapeDtypeStruct((b, 1, c3), F32),
        in_specs=[pl.BlockSpec((1, n, cp), lambda bi: (bi, 0, 0))] + wspecs,
        out_specs=pl.BlockSpec((1, 1, c3), lambda bi: (bi, 0, 0)),
    )(xf, w1, s1, b1, w2, s2, b2, w3, s3, b3)
    return jnp.squeeze(out, axis=1)


def _prep(layers):
    eps = 1e-5
    out = []
    for (w, gamma, beta) in layers:
        s = (gamma / jnp.sqrt(1.0 + eps)).reshape(1, -1)
        out.append((w.T.astype(jnp.bfloat16), s, beta.reshape(1, -1)))
    return out


def kernel(pointcloud, params):
    pc = pointcloud.astype(F32)
    xyz = pc[..., 0:3]
    xt1 = jnp.transpose(xyz, (0, 2, 1))
    xf1 = pc

    c1 = _fps(xt1, 512)
    nx1 = jnp.transpose(c1, (0, 2, 1))
    rad1, ns1 = [0.1, 0.2, 0.4], [16, 32, 128]
    outs1 = [_sa(xt1, xf1, nx1, _prep(params[0][j]), rad1[j] * rad1[j],
                 ns1[j], sb=8, cw=2048) for j in range(3)]
    xf2 = jnp.concatenate([nx1] + outs1, axis=-1)

    c2 = _fps(c1, 128)
    nx2 = jnp.transpose(c2, (0, 2, 1))
    rad2, ns2 = [0.2, 0.4, 0.8], [32, 64, 128]
    outs2 = [_sa(c1, xf2, nx2, _prep(params[1][j]), rad2[j] * rad2[j],
                 ns2[j], sb=8, cw=512) for j in range(3)]
    xf3 = jnp.concatenate([nx2] + outs2, axis=-1)

    return _ga(xf3, _prep(params[2][0]))


# speedup vs baseline: 7.8702x; 3.9439x over previous
import functools

import jax
import jax.numpy as jnp
from jax.experimental import pallas as pl

F32 = jnp.float32


def _fiota(shape, dim):
    return jax.lax.broadcasted_iota(jnp.int32, shape, dim).astype(F32)


def _fps_kernel(xt_ref, out_ref, *, npoint, n):
    b = xt_ref.shape[0]
    niota = _fiota((b, n), 1)
    piota = jax.lax.broadcasted_iota(jnp.int32, (b, npoint), 1)
    x0 = xt_ref[:, 0, :]
    x1 = xt_ref[:, 1, :]
    x2 = xt_ref[:, 2, :]

    def body(i, carry):
        dist, oh, cx, cy, cz = carry
        c0 = jnp.sum(x0 * oh, axis=1, keepdims=True)
        c1 = jnp.sum(x1 * oh, axis=1, keepdims=True)
        c2 = jnp.sum(x2 * oh, axis=1, keepdims=True)
        sel = (piota == i).astype(F32)
        cx = cx + c0 * sel
        cy = cy + c1 * sel
        cz = cz + c2 * sel
        d = (x0 - c0) ** 2 + (x1 - c1) ** 2 + (x2 - c2) ** 2
        dist = jnp.minimum(dist, d)
        m = jnp.max(dist, axis=1, keepdims=True)
        idx = jnp.min(jnp.where(dist == m, niota, float(n)), axis=1,
                      keepdims=True)
        oh = (niota == idx).astype(F32)
        return dist, oh, cx, cy, cz

    dist0 = jnp.full((b, n), 1e10, F32)
    oh0 = (niota == 0.0).astype(F32)
    z = jnp.zeros((b, npoint), F32)
    _, _, cx, cy, cz = jax.lax.fori_loop(0, npoint, body,
                                         (dist0, oh0, z, z, z))
    out_ref[:, 0, :] = cx
    out_ref[:, 1, :] = cy
    out_ref[:, 2, :] = cz


def _fps(xt, npoint):
    b, _, n = xt.shape
    return pl.pallas_call(
        functools.partial(_fps_kernel, npoint=npoint, n=n),
        out_shape=jax.ShapeDtypeStruct((b, 3, npoint), F32),
        in_specs=[pl.BlockSpec((b, 3, n), lambda: (0, 0, 0))],
        out_specs=pl.BlockSpec((b, 3, npoint), lambda: (0, 0, 0)),
    )(xt)


def _mlp3(x, w1, s1, b1, w2, s2, b2, w3, s3, b3):
    for w, s, b in ((w1, s1, b1), (w2, s2, b2), (w3, s3, b3)):
        x = jnp.dot(x.astype(jnp.bfloat16), w[...],
                    preferred_element_type=F32)
        x = jnp.maximum(x * s[...] + b[...], 0.0)
    return x


def _sa_kernel(xt_ref, xf_ref, nx_ref, w1_ref, s1_ref, b1_ref, w2_ref,
               s2_ref, b2_ref, w3_ref, s3_ref, b3_ref, o_ref,
               *, r2, k, n, sb, cw):
    nc = n // 128
    xt = xt_ref[0]
    nx = nx_ref[0]
    aa = nx[:, 0:1] ** 2 + nx[:, 1:2] ** 2 + nx[:, 2:3] ** 2
    bb = xt[0:1, :] ** 2 + xt[1:2, :] ** 2 + xt[2:3, :] ** 2
    ab = jnp.dot(nx.astype(jnp.bfloat16), xt.astype(jnp.bfloat16),
                 preferred_element_type=F32)
    d2 = aa + bb - 2.0 * ab
    m = (d2 < r2).astype(F32)

    ii = _fiota((128, 128), 0)
    jj = _fiota((128, 128), 1)
    tri = (ii <= jj).astype(F32)
    incl, tots = [], []
    for c in range(nc):
        mc = m[:, 128 * c:128 * (c + 1)]
        ic = jnp.dot(mc, tri, preferred_element_type=F32)
        incl.append(ic)
        tots.append(ic[:, 127:128])
    totals = jnp.concatenate(tots, axis=1)
    si = _fiota((nc, nc), 0)
    sj = _fiota((nc, nc), 1)
    stri = (si < sj).astype(F32)
    offs = jnp.dot(totals, stri, preferred_element_type=F32)
    rank = jnp.concatenate(
        [incl[c] - m[:, 128 * c:128 * (c + 1)] + offs[:, c:c + 1]
         for c in range(nc)], axis=1)
    counts = offs[:, nc - 1:nc] + totals[:, nc - 1:nc]

    kio = _fiota((k, 1), 0)
    cp = xf_ref.shape[2]
    nxpad = jnp.concatenate([nx, jnp.zeros((sb, cp - 3), F32)], axis=1)
    rmk = jnp.where(m > 0.0, rank, 1e9)
    g_rows = []
    for s in range(sb):
        cnt = counts[s:s + 1, 0:1]
        keff = jnp.minimum(kio, cnt - 1.0)
        rs = rmk[s:s + 1, :]
        g = jnp.zeros((k, cp), F32)
        for c0 in range(0, n, cw):
            e = (rs[:, c0:c0 + cw] == keff).astype(F32)
            if c0 == 0:
                oh0 = (_fiota((1, cw), 1)
                       == 0.0).astype(F32)
                e = jnp.where(cnt > 0.0, e, oh0)
            g = g + jnp.dot(e, xf_ref[0, c0:c0 + cw, :],
                            preferred_element_type=F32)
        g_rows.append(g - nxpad[s:s + 1, :])
    gall = jnp.concatenate(g_rows, axis=0)

    a = _mlp3(gall, w1_ref, s1_ref, b1_ref, w2_ref, s2_ref, b2_ref,
              w3_ref, s3_ref, b3_ref)
    for s in range(sb):
        o_ref[0, s:s + 1, :] = jnp.max(a[s * k:(s + 1) * k, :], axis=0,
                                       keepdims=True)


def _sa(xt, xf, nx, wts, r2, k, sb, cw):
    b, _, n = xt.shape
    s_tot = nx.shape[1]
    cp = xf.shape[2]
    (w1, s1, b1), (w2, s2, b2), (w3, s3, b3) = wts
    c3 = w3.shape[1]
    wspecs = [pl.BlockSpec(a.shape, lambda bi, st: (0, 0))
              for a in (w1, s1, b1, w2, s2, b2, w3, s3, b3)]
    return pl.pallas_call(
        functools.partial(_sa_kernel, r2=r2, k=k, n=n, sb=sb, cw=cw),
        grid=(b, s_tot // sb),
        out_shape=jax.ShapeDtypeStruct((b, s_tot, c3), F32),
        in_specs=[
            pl.BlockSpec((1, 3, n), lambda bi, st: (bi, 0, 0)),
            pl.BlockSpec((1, n, cp), lambda bi, st: (bi, 0, 0)),
            pl.BlockSpec((1, sb, 3), lambda bi, st: (bi, st, 0)),
        ] + wspecs,
        out_specs=pl.BlockSpec((1, sb, c3), lambda bi, st: (bi, st, 0)),
    )(xt, xf, nx, w1, s1, b1, w2, s2, b2, w3, s3, b3)


def _ga_kernel(xf_ref, w1_ref, s1_ref, b1_ref, w2_ref, s2_ref, b2_ref,
               w3_ref, s3_ref, b3_ref, o_ref):
    a = _mlp3(xf_ref[0], w1_ref, s1_ref, b1_ref, w2_ref, s2_ref, b2_ref,
              w3_ref, s3_ref, b3_ref)
    o_ref[0] = jnp.max(a, axis=0, keepdims=True)


def _ga(xf, wts):
    b, n, cp = xf.shape
    (w1, s1, b1), (w2, s2, b2), (w3, s3, b3) = wts
    c3 = w3.shape[1]
    wspecs = [pl.BlockSpec(a.shape, lambda bi: (0, 0))
              for a in (w1, s1, b1, w2, s2, b2, w3, s3, b3)]
    out = pl.pallas_call(
        _ga_kernel,
        grid=(b,),
        out_shape=jax.ShapeDtypeStruct((b, 1, c3), F32),
        in_specs=[pl.BlockSpec((1, n, cp), lambda bi: (bi, 0, 0))] + wspecs,
        out_specs=pl.BlockSpec((1, 1, c3), lambda bi: (bi, 0, 0)),
    )(xf, w1, s1, b1, w2, s2, b2, w3, s3, b3)
    return jnp.squeeze(out, axis=1)


def _prep(layers):
    eps = 1e-5
    out = []
    for (w, gamma, beta) in layers:
        s = (gamma / jnp.sqrt(1.0 + eps)).reshape(1, -1)
        out.append((w.T.astype(jnp.bfloat16), s, beta.reshape(1, -1)))
    return out


def kernel(pointcloud, params):
    pc = pointcloud.astype(F32)
    xyz = pc[..., 0:3]
    xt1 = jnp.transpose(xyz, (0, 2, 1))
    xf1 = pc

    c1 = _fps(xt1, 512)
    nx1 = jnp.transpose(c1, (0, 2, 1))
    rad1, ns1 = [0.1, 0.2, 0.4], [16, 32, 128]
    outs1 = [_sa(xt1, xf1, nx1, _prep(params[0][j]), rad1[j] * rad1[j],
                 ns1[j], sb=8, cw=2048) for j in range(3)]
    xf2 = jnp.concatenate([nx1] + outs1, axis=-1)

    c2 = _fps(c1, 128)
    nx2 = jnp.transpose(c2, (0, 2, 1))
    rad2, ns2 = [0.2, 0.4, 0.8], [32, 64, 128]
    outs2 = [_sa(c1, xf2, nx2, _prep(params[1][j]), rad2[j] * rad2[j],
                 ns2[j], sb=8, cw=512) for j in range(3)]
    xf3 = jnp.concatenate([nx2] + outs2, axis=-1)

    return _ga(xf3, _prep(params[2][0]))
